# baseline (speedup 1.0000x reference)
import functools

import jax
import jax.numpy as jnp
from jax import lax
from jax.experimental import pallas as pl
from jax.experimental.pallas import tpu as pltpu
from jax.experimental.pallas import tpu_sc as plsc

N_NODES = 10000
P = 5000
C = 64
D = 128
ATT = 64

_NC, _NS = 2, 16
_NW = _NC * _NS
_CHUNK = 80
_NBUF = 5
_PA = 3000
_PB = P - _PA


def _sc_gather_body(per_w, table_hbm, idx_hbm, out_hbm, idx_v, *bufs):
    niter = per_w // _CHUNK
    rows = bufs[:_NBUF]
    gsems = bufs[_NBUF:2 * _NBUF]
    wsems = bufs[2 * _NBUF:]
    wid = lax.axis_index("s") * _NC + lax.axis_index("c")
    base0 = pl.multiple_of(wid * per_w, 8)
    pltpu.sync_copy(idx_hbm.at[pl.ds(base0, per_w)], idx_v)

    def _start_gather(g, k):
        off = pl.multiple_of(g * _CHUNK, 8)
        pltpu.async_copy(table_hbm.at[idx_v.at[pl.ds(off, _CHUNK)]],
                         rows[k], gsems[k])

    def _finish(g, k):
        pltpu.make_async_copy(table_hbm.at[idx_v.at[pl.ds(0, _CHUNK)]],
                              rows[k], gsems[k]).wait()
        out_off = pl.multiple_of(base0 + g * _CHUNK, 8)
        pltpu.async_copy(rows[k], out_hbm.at[pl.ds(out_off, _CHUNK)],
                         wsems[k])

    def _wait_write(k):
        pltpu.make_async_copy(rows[k],
                              out_hbm.at[pl.ds(base0, _CHUNK)],
                              wsems[k]).wait()

    for k in range(_NBUF):
        _start_gather(k, k)

    def body(jj, _):
        g0 = _NBUF * jj
        for k in range(_NBUF):
            _finish(g0 + k, k)
        for k in range(_NBUF):
            _wait_write(k)
            _start_gather(g0 + _NBUF + k, k)
        return ()

    lax.fori_loop(0, niter // _NBUF - 1, body, ())
    g0 = niter - _NBUF
    for k in range(_NBUF):
        _finish(g0 + k, k)
    for k in range(_NBUF):
        _wait_write(k)


def _sc_gather(table, idx_flat, n_parents):
    total = n_parents * C
    per_w = total // _NW
    mesh = plsc.VectorSubcoreMesh(core_axis_name="c", subcore_axis_name="s")
    f = functools.partial(
        pl.kernel,
        mesh=mesh,
        out_type=jax.ShapeDtypeStruct((total, D), jnp.float32),
        scratch_types=(
            [pltpu.VMEM((per_w,), jnp.int32)]
            + [pltpu.VMEM((_CHUNK, D), jnp.float32) for _ in range(_NBUF)]
            + [pltpu.SemaphoreType.DMA for _ in range(2 * _NBUF)]
        ),
    )(functools.partial(_sc_gather_body, per_w))
    return f(table, idx_flat)


_BP = 200


def _tc_dense_body(wpoi_ref, gath_ref, attw_ref, b_ref, v_ref, mask_ref,
                   out_ref, s_ref):
    wp = wpoi_ref[...]
    top = attw_ref[:D, :]
    bot = attw_ref[D:, :]
    pp = jnp.dot(wp, top, preferred_element_type=jnp.float32)
    pp = pp + b_ref[...]
    xf = gath_ref[...]
    xall = xf.reshape(C * _BP, D)
    cp = jnp.dot(xall, bot, preferred_element_type=jnp.float32)
    pptile = jnp.broadcast_to(pp[None], (C, _BP, ATT)).reshape(C * _BP, ATT)
    pre = jnp.tanh(cp + pptile)
    sflat = jnp.dot(pre, v_ref[...],
                    preferred_element_type=jnp.float32)
    for c in range(C):
        s_ref[:, c:c + 1] = sflat[c * _BP:(c + 1) * _BP]
    att = jax.nn.softmax(s_ref[...] + mask_ref[...], axis=1)
    acc = xf[0] * att[:, 0:1]
    for c in range(1, C):
        acc = acc + xf[c] * att[:, c:c + 1]
    out_ref[...] = acc


def _tc_dense(W_poi, gathered, att_W, att_b, v_attention, mask, p0, np_):
    nblk = np_ // _BP
    blk0 = p0 // _BP
    return pl.pallas_call(
        _tc_dense_body,
        grid=(nblk,),
        in_specs=[
            pl.BlockSpec((_BP, D), lambda i: (i + blk0, 0)),
            pl.BlockSpec((C, _BP, D), lambda i: (0, i, 0)),
            pl.BlockSpec((2 * D, ATT), lambda i: (0, 0)),
            pl.BlockSpec((1, ATT), lambda i: (0, 0)),
            pl.BlockSpec((ATT, 1), lambda i: (0, 0)),
            pl.BlockSpec((_BP, C), lambda i: (i + blk0, 0)),
        ],
        out_specs=pl.BlockSpec((_BP, D), lambda i: (i + blk0, 0)),
        out_shape=jax.ShapeDtypeStruct((N_NODES, D), jnp.float32),
        scratch_shapes=[pltpu.VMEM((_BP, C), jnp.float32)],
    )(W_poi, gathered, att_W, att_b, v_attention, mask)


_BC = 1000


def _tc_copy_body(w_ref, acc_ref, out_ref):
    out_ref[...] = w_ref[...]


def _tc_copy_into(W_poi, dense_out):
    nblk = (N_NODES - P) // _BC
    off = P // _BC
    return pl.pallas_call(
        _tc_copy_body,
        grid=(nblk,),
        in_specs=[
            pl.BlockSpec((_BC, D), lambda i: (i + off, 0)),
            pl.BlockSpec(memory_space=pl.ANY),
        ],
        out_specs=pl.BlockSpec((_BC, D), lambda i: (i + off, 0)),
        out_shape=jax.ShapeDtypeStruct((N_NODES, D), jnp.float32),
        input_output_aliases={1: 0},
    )(W_poi, dense_out)


def kernel(W_poi, att_W, att_b, v_attention, mask, parent_ids, children):
    ch = children.astype(jnp.int32)
    idx_flat = jnp.transpose(ch).reshape(-1)
    b2 = att_b.reshape(1, ATT)
    v2 = v_attention.reshape(ATT, 1)

    gath = _sc_gather(W_poi, idx_flat, P).reshape(C, P, D)
    dense = _tc_dense(W_poi, gath, att_W, b2, v2, mask, 0, P)
    return _tc_copy_into(W_poi, dense)


# speedup vs baseline: 1.0931x; 1.0931x over previous
import functools

import jax
import jax.numpy as jnp
from jax import lax
from jax.experimental import pallas as pl
from jax.experimental.pallas import tpu as pltpu
from jax.experimental.pallas import tpu_sc as plsc

N_NODES = 10000
P = 5000
C = 64
D = 128
ATT = 64

_NC, _NS = 2, 16
_NW = _NC * _NS
_CHUNK = 80
_NBUF = 5
_PA = 3000
_PB = P - _PA


def _sc_gather_body(per_w, table_hbm, idx_hbm, out_hbm, idx_v, *bufs):
    niter = per_w // _CHUNK
    rows = bufs[:_NBUF]
    gsems = bufs[_NBUF:2 * _NBUF]
    wsems = bufs[2 * _NBUF:]
    wid = lax.axis_index("s") * _NC + lax.axis_index("c")
    base0 = pl.multiple_of(wid * per_w, 8)
    pltpu.sync_copy(idx_hbm.at[pl.ds(base0, per_w)], idx_v)

    def _start_gather(g, k):
        off = pl.multiple_of(g * _CHUNK, 8)
        pltpu.async_copy(table_hbm.at[idx_v.at[pl.ds(off, _CHUNK)]],
                         rows[k], gsems[k])

    def _finish(g, k):
        pltpu.make_async_copy(table_hbm.at[idx_v.at[pl.ds(0, _CHUNK)]],
                              rows[k], gsems[k]).wait()
        out_off = pl.multiple_of(base0 + g * _CHUNK, 8)
        pltpu.async_copy(rows[k], out_hbm.at[pl.ds(out_off, _CHUNK)],
                         wsems[k])

    def _wait_write(k):
        pltpu.make_async_copy(rows[k],
                              out_hbm.at[pl.ds(base0, _CHUNK)],
                              wsems[k]).wait()

    for k in range(_NBUF):
        _start_gather(k, k)

    def body(jj, _):
        g0 = _NBUF * jj
        for k in range(_NBUF):
            _finish(g0 + k, k)
        for k in range(_NBUF):
            _wait_write(k)
            _start_gather(g0 + _NBUF + k, k)
        return ()

    lax.fori_loop(0, niter // _NBUF - 1, body, ())
    g0 = niter - _NBUF
    for k in range(_NBUF):
        _finish(g0 + k, k)
    for k in range(_NBUF):
        _wait_write(k)


def _sc_gather(table, idx_flat, n_parents):
    total = n_parents * C
    per_w = total // _NW
    mesh = plsc.VectorSubcoreMesh(core_axis_name="c", subcore_axis_name="s")
    f = functools.partial(
        pl.kernel,
        mesh=mesh,
        out_type=jax.ShapeDtypeStruct((total, D), jnp.float32),
        scratch_types=(
            [pltpu.VMEM((per_w,), jnp.int32)]
            + [pltpu.VMEM((_CHUNK, D), jnp.float32) for _ in range(_NBUF)]
            + [pltpu.SemaphoreType.DMA for _ in range(2 * _NBUF)]
        ),
    )(functools.partial(_sc_gather_body, per_w))
    return f(table, idx_flat)


_BP = 200


def _tc_dense_body(wpoi_ref, gath_ref, attw_ref, b_ref, v_ref, mask_ref,
                   *rest):
    out_ref, s_ref = rest[-2:]
    wp = wpoi_ref[...]
    top = attw_ref[:D, :]
    bot = attw_ref[D:, :]
    pp = jnp.dot(wp, top, preferred_element_type=jnp.float32)
    pp = pp + b_ref[...]
    xf = gath_ref[...]
    xall = xf.reshape(C * _BP, D)
    cp = jnp.dot(xall, bot, preferred_element_type=jnp.float32)
    pptile = jnp.broadcast_to(pp[None], (C, _BP, ATT)).reshape(C * _BP, ATT)
    pre = jnp.tanh(cp + pptile)
    sflat = jnp.dot(pre, v_ref[...],
                    preferred_element_type=jnp.float32)
    for c in range(C):
        s_ref[:, c:c + 1] = sflat[c * _BP:(c + 1) * _BP]
    att = jax.nn.softmax(s_ref[...] + mask_ref[...], axis=1)
    acc = xf[0] * att[:, 0:1]
    for c in range(1, C):
        acc = acc + xf[c] * att[:, c:c + 1]
    out_ref[...] = acc


def _tc_dense(W_poi, gathered, att_W, att_b, v_attention, mask, p0, np_,
              carry=None):
    nblk = np_ // _BP
    blk0 = p0 // _BP
    in_specs = [
        pl.BlockSpec((_BP, D), lambda i: (i + blk0, 0)),
        pl.BlockSpec((C, _BP, D), lambda i: (0, i, 0)),
        pl.BlockSpec((2 * D, ATT), lambda i: (0, 0)),
        pl.BlockSpec((1, ATT), lambda i: (0, 0)),
        pl.BlockSpec((ATT, 1), lambda i: (0, 0)),
        pl.BlockSpec((_BP, C), lambda i: (i + blk0, 0)),
    ]
    args = [W_poi, gathered, att_W, att_b, v_attention, mask]
    aliases = {}
    if carry is not None:
        in_specs.append(pl.BlockSpec(memory_space=pl.ANY))
        args.append(carry)
        aliases = {6: 0}
    return pl.pallas_call(
        _tc_dense_body,
        grid=(nblk,),
        in_specs=in_specs,
        out_specs=pl.BlockSpec((_BP, D), lambda i: (i + blk0, 0)),
        out_shape=jax.ShapeDtypeStruct((N_NODES, D), jnp.float32),
        scratch_shapes=[pltpu.VMEM((_BP, C), jnp.float32)],
        input_output_aliases=aliases,
    )(*args)


_BC = 1000


def _tc_copy_body(w_ref, acc_ref, out_ref):
    out_ref[...] = w_ref[...]


def _tc_copy_into(W_poi, dense_out):
    nblk = (N_NODES - P) // _BC
    off = P // _BC
    return pl.pallas_call(
        _tc_copy_body,
        grid=(nblk,),
        in_specs=[
            pl.BlockSpec((_BC, D), lambda i: (i + off, 0)),
            pl.BlockSpec(memory_space=pl.ANY),
        ],
        out_specs=pl.BlockSpec((_BC, D), lambda i: (i + off, 0)),
        out_shape=jax.ShapeDtypeStruct((N_NODES, D), jnp.float32),
        input_output_aliases={1: 0},
    )(W_poi, dense_out)


def kernel(W_poi, att_W, att_b, v_attention, mask, parent_ids, children):
    ch = children.astype(jnp.int32)
    idx_a = jnp.transpose(ch[:_PA]).reshape(-1)
    idx_b = jnp.transpose(ch[_PA:]).reshape(-1)
    b2 = att_b.reshape(1, ATT)
    v2 = v_attention.reshape(ATT, 1)

    gath_a = _sc_gather(W_poi, idx_a, _PA).reshape(C, _PA, D)
    gath_b = _sc_gather(W_poi, idx_b, _PB).reshape(C, _PB, D)
    out_a = _tc_dense(W_poi, gath_a, att_W, b2, v2, mask, 0, _PA)
    out_b = _tc_dense(W_poi, gath_b, att_W, b2, v2, mask, _PA, _PB,
                      carry=out_a)
    return _tc_copy_into(W_poi, out_b)


# speedup vs baseline: 1.1594x; 1.0607x over previous
import functools

import jax
import jax.numpy as jnp
from jax import lax
from jax.experimental import pallas as pl
from jax.experimental.pallas import tpu as pltpu
from jax.experimental.pallas import tpu_sc as plsc

N_NODES = 10000
P = 5000
C = 64
D = 128
ATT = 64

_NC, _NS = 2, 16
_NW = _NC * _NS
_CHUNK = 80
_NBUF = 5
_PA = 3000
_PB = P - _PA


def _sc_gather_body(per_w, table_hbm, idx_hbm, out_hbm, idx_v, *bufs):
    niter = per_w // _CHUNK
    rows = bufs[:_NBUF]
    gsems = bufs[_NBUF:2 * _NBUF]
    wsems = bufs[2 * _NBUF:]
    wid = lax.axis_index("s") * _NC + lax.axis_index("c")
    base0 = pl.multiple_of(wid * per_w, 8)
    pltpu.sync_copy(idx_hbm.at[pl.ds(base0, per_w)], idx_v)

    def _start_gather(g, k):
        off = pl.multiple_of(g * _CHUNK, 8)
        pltpu.async_copy(table_hbm.at[idx_v.at[pl.ds(off, _CHUNK)]],
                         rows[k], gsems[k])

    def _finish(g, k):
        pltpu.make_async_copy(table_hbm.at[idx_v.at[pl.ds(0, _CHUNK)]],
                              rows[k], gsems[k]).wait()
        out_off = pl.multiple_of(base0 + g * _CHUNK, 8)
        pltpu.async_copy(rows[k], out_hbm.at[pl.ds(out_off, _CHUNK)],
                         wsems[k])

    def _wait_write(k):
        pltpu.make_async_copy(rows[k],
                              out_hbm.at[pl.ds(base0, _CHUNK)],
                              wsems[k]).wait()

    for k in range(_NBUF):
        _start_gather(k, k)

    def body(jj, _):
        g0 = _NBUF * jj
        for k in range(_NBUF):
            _finish(g0 + k, k)
        for k in range(_NBUF):
            _wait_write(k)
            _start_gather(g0 + _NBUF + k, k)
        return ()

    lax.fori_loop(0, niter // _NBUF - 1, body, ())
    g0 = niter - _NBUF
    for k in range(_NBUF):
        _finish(g0 + k, k)
    for k in range(_NBUF):
        _wait_write(k)


def _sc_gather(table, idx_flat, n_parents):
    total = n_parents * C
    per_w = total // _NW
    mesh = plsc.VectorSubcoreMesh(core_axis_name="c", subcore_axis_name="s")
    f = functools.partial(
        pl.kernel,
        mesh=mesh,
        out_type=jax.ShapeDtypeStruct((total, D), jnp.float32),
        scratch_types=(
            [pltpu.VMEM((per_w,), jnp.int32)]
            + [pltpu.VMEM((_CHUNK, D), jnp.float32) for _ in range(_NBUF)]
            + [pltpu.SemaphoreType.DMA for _ in range(2 * _NBUF)]
        ),
    )(functools.partial(_sc_gather_body, per_w))
    return f(table, idx_flat)


_BP = 200


def _tc_dense_body(wpoi_ref, gath_ref, attw_ref, b_ref, v_ref, mask_ref,
                   eye_ref, *rest):
    out_ref, s_ref = rest[-2:]
    wp = wpoi_ref[...]
    top = attw_ref[:D, :]
    bot = attw_ref[D:, :]
    pp = jnp.dot(wp, top, preferred_element_type=jnp.float32)
    pp = pp + b_ref[...]
    xf = gath_ref[...]
    xall = xf.reshape(C * _BP, D)
    cp = jnp.dot(xall, bot, preferred_element_type=jnp.float32)
    pptile = jnp.broadcast_to(pp[None], (C, _BP, ATT)).reshape(C * _BP, ATT)
    pre = jnp.tanh(cp + pptile)
    sflat = jnp.dot(pre, v_ref[...],
                    preferred_element_type=jnp.float32)
    for c in range(C):
        s_ref[:, c:c + 1] = sflat[c * _BP:(c + 1) * _BP]
    att = jax.nn.softmax(s_ref[...] + mask_ref[...], axis=1)
    bcb = jnp.dot(att, eye_ref[...],
                  preferred_element_type=jnp.float32)
    acc = xf[0] * bcb[:, :D]
    for c in range(1, C):
        acc = acc + xf[c] * bcb[:, c * D:(c + 1) * D]
    out_ref[...] = acc


def _tc_dense(W_poi, gathered, att_W, att_b, v_attention, mask, eye_blk,
              p0, np_, carry=None):
    nblk = np_ // _BP
    blk0 = p0 // _BP
    in_specs = [
        pl.BlockSpec((_BP, D), lambda i: (i + blk0, 0)),
        pl.BlockSpec((C, _BP, D), lambda i: (0, i, 0)),
        pl.BlockSpec((2 * D, ATT), lambda i: (0, 0)),
        pl.BlockSpec((1, ATT), lambda i: (0, 0)),
        pl.BlockSpec((ATT, 1), lambda i: (0, 0)),
        pl.BlockSpec((_BP, C), lambda i: (i + blk0, 0)),
        pl.BlockSpec((C, C * D), lambda i: (0, 0)),
    ]
    args = [W_poi, gathered, att_W, att_b, v_attention, mask, eye_blk]
    aliases = {}
    if carry is not None:
        in_specs.append(pl.BlockSpec(memory_space=pl.ANY))
        args.append(carry)
        aliases = {7: 0}
    return pl.pallas_call(
        _tc_dense_body,
        grid=(nblk,),
        in_specs=in_specs,
        out_specs=pl.BlockSpec((_BP, D), lambda i: (i + blk0, 0)),
        out_shape=jax.ShapeDtypeStruct((N_NODES, D), jnp.float32),
        scratch_shapes=[pltpu.VMEM((_BP, C), jnp.float32)],
        input_output_aliases=aliases,
    )(*args)


_BC = 1000


def _tc_copy_body(w_ref, acc_ref, out_ref):
    out_ref[...] = w_ref[...]


def _tc_copy_into(W_poi, dense_out):
    nblk = (N_NODES - P) // _BC
    off = P // _BC
    return pl.pallas_call(
        _tc_copy_body,
        grid=(nblk,),
        in_specs=[
            pl.BlockSpec((_BC, D), lambda i: (i + off, 0)),
            pl.BlockSpec(memory_space=pl.ANY),
        ],
        out_specs=pl.BlockSpec((_BC, D), lambda i: (i + off, 0)),
        out_shape=jax.ShapeDtypeStruct((N_NODES, D), jnp.float32),
        input_output_aliases={1: 0},
    )(W_poi, dense_out)


def kernel(W_poi, att_W, att_b, v_attention, mask, parent_ids, children):
    ch = children.astype(jnp.int32)
    idx_a = jnp.transpose(ch[:_PA]).reshape(-1)
    idx_b = jnp.transpose(ch[_PA:]).reshape(-1)
    b2 = att_b.reshape(1, ATT)
    v2 = v_attention.reshape(ATT, 1)
    eye_blk = jnp.repeat(jnp.eye(C, dtype=jnp.float32), D, axis=1)

    gath_a = _sc_gather(W_poi, idx_a, _PA).reshape(C, _PA, D)
    gath_b = _sc_gather(W_poi, idx_b, _PB).reshape(C, _PB, D)
    out_a = _tc_dense(W_poi, gath_a, att_W, b2, v2, mask, eye_blk, 0, _PA)
    out_b = _tc_dense(W_poi, gath_b, att_W, b2, v2, mask, eye_blk, _PA, _PB,
                      carry=out_a)
    return _tc_copy_into(W_poi, out_b)


# speedup vs baseline: 1.1781x; 1.0161x over previous
import functools

import jax
import jax.numpy as jnp
from jax import lax
from jax.experimental import pallas as pl
from jax.experimental.pallas import tpu as pltpu
from jax.experimental.pallas import tpu_sc as plsc

N_NODES = 10000
P = 5000
C = 64
D = 128
ATT = 64

_NC, _NS = 2, 16
_NW = _NC * _NS
_CHUNK = 80
_NBUF = 5
_PA = 3000
_PB = P - _PA


def _sc_gather_body(per_w, table_hbm, idx_hbm, out_hbm, idx_v, *bufs):
    niter = per_w // _CHUNK
    rows = bufs[:_NBUF]
    gsems = bufs[_NBUF:2 * _NBUF]
    wsems = bufs[2 * _NBUF:]
    wid = lax.axis_index("s") * _NC + lax.axis_index("c")
    base0 = pl.multiple_of(wid * per_w, 8)
    pltpu.sync_copy(idx_hbm.at[pl.ds(base0, per_w)], idx_v)

    def _start_gather(g, k):
        off = pl.multiple_of(g * _CHUNK, 8)
        pltpu.async_copy(table_hbm.at[idx_v.at[pl.ds(off, _CHUNK)]],
                         rows[k], gsems[k])

    def _finish(g, k):
        pltpu.make_async_copy(table_hbm.at[idx_v.at[pl.ds(0, _CHUNK)]],
                              rows[k], gsems[k]).wait()
        out_off = pl.multiple_of(base0 + g * _CHUNK, 8)
        pltpu.async_copy(rows[k], out_hbm.at[pl.ds(out_off, _CHUNK)],
                         wsems[k])

    def _wait_write(k):
        pltpu.make_async_copy(rows[k],
                              out_hbm.at[pl.ds(base0, _CHUNK)],
                              wsems[k]).wait()

    for k in range(_NBUF):
        _start_gather(k, k)

    def body(jj, _):
        g0 = _NBUF * jj
        for k in range(_NBUF):
            _finish(g0 + k, k)
        for k in range(_NBUF):
            _wait_write(k)
            _start_gather(g0 + _NBUF + k, k)
        return ()

    lax.fori_loop(0, niter // _NBUF - 1, body, ())
    g0 = niter - _NBUF
    for k in range(_NBUF):
        _finish(g0 + k, k)
    for k in range(_NBUF):
        _wait_write(k)


def _sc_gather(table, idx_flat, n_parents):
    total = n_parents * C
    per_w = total // _NW
    mesh = plsc.VectorSubcoreMesh(core_axis_name="c", subcore_axis_name="s")
    f = functools.partial(
        pl.kernel,
        mesh=mesh,
        out_type=jax.ShapeDtypeStruct((total, D), jnp.float32),
        scratch_types=(
            [pltpu.VMEM((per_w,), jnp.int32)]
            + [pltpu.VMEM((_CHUNK, D), jnp.float32) for _ in range(_NBUF)]
            + [pltpu.SemaphoreType.DMA for _ in range(2 * _NBUF)]
        ),
    )(functools.partial(_sc_gather_body, per_w))
    return f(table, idx_flat)


_BP = 200


def _tc_dense_body(wpoi_ref, gath_ref, attw_ref, b_ref, v_ref, mask_ref,
                   eye_ref, *rest):
    out_ref, s_ref = rest[-2:]
    wp = wpoi_ref[...]
    top = attw_ref[:D, :]
    bot = attw_ref[D:, :]
    pp = jnp.dot(wp, top, preferred_element_type=jnp.float32)
    pp = pp + b_ref[...]
    xf = gath_ref[...]
    xall = xf.reshape(C * _BP, D)
    cp = jnp.dot(xall, bot, preferred_element_type=jnp.float32)
    pptile = jnp.broadcast_to(pp[None], (C, _BP, ATT)).reshape(C * _BP, ATT)
    pre = jnp.tanh(cp + pptile)
    sflat = jnp.dot(pre, v_ref[...],
                    preferred_element_type=jnp.float32)
    for c in range(C):
        s_ref[:, c:c + 1] = sflat[c * _BP:(c + 1) * _BP]
    att = jax.nn.softmax(s_ref[...] + mask_ref[...], axis=1)
    bcb = jnp.dot(att, eye_ref[...],
                  preferred_element_type=jnp.float32)
    acc = xf[0] * bcb[:, :D]
    for c in range(1, C):
        acc = acc + xf[c] * bcb[:, c * D:(c + 1) * D]
    out_ref[...] = acc


def _tc_dense(W_poi, gathered, att_W, att_b, v_attention, mask, eye_blk,
              p0, np_, carry=None):
    nblk = np_ // _BP
    blk0 = p0 // _BP
    in_specs = [
        pl.BlockSpec((_BP, D), lambda i: (i + blk0, 0)),
        pl.BlockSpec((C, _BP, D), lambda i: (0, i, 0)),
        pl.BlockSpec((2 * D, ATT), lambda i: (0, 0)),
        pl.BlockSpec((1, ATT), lambda i: (0, 0)),
        pl.BlockSpec((ATT, 1), lambda i: (0, 0)),
        pl.BlockSpec((_BP, C), lambda i: (i + blk0, 0)),
        pl.BlockSpec((C, C * D), lambda i: (0, 0)),
    ]
    args = [W_poi, gathered, att_W, att_b, v_attention, mask, eye_blk]
    aliases = {}
    if carry is not None:
        in_specs.append(pl.BlockSpec(memory_space=pl.ANY))
        args.append(carry)
        aliases = {7: 0}
    return pl.pallas_call(
        _tc_dense_body,
        grid=(nblk,),
        in_specs=in_specs,
        out_specs=pl.BlockSpec((_BP, D), lambda i: (i + blk0, 0)),
        out_shape=jax.ShapeDtypeStruct((N_NODES, D), jnp.float32),
        scratch_shapes=[pltpu.VMEM((_BP, C), jnp.float32)],
        input_output_aliases=aliases,
    )(*args)


_BC = 1000


def _tc_copy_body(w_ref, acc_ref, out_ref):
    out_ref[...] = w_ref[...]


def _tc_copy_into(W_poi, dense_out):
    nblk = (N_NODES - P) // _BC
    off = P // _BC
    return pl.pallas_call(
        _tc_copy_body,
        grid=(nblk,),
        in_specs=[
            pl.BlockSpec((_BC, D), lambda i: (i + off, 0)),
            pl.BlockSpec(memory_space=pl.ANY),
        ],
        out_specs=pl.BlockSpec((_BC, D), lambda i: (i + off, 0)),
        out_shape=jax.ShapeDtypeStruct((N_NODES, D), jnp.float32),
        input_output_aliases={1: 0},
    )(W_poi, dense_out)


def kernel(W_poi, att_W, att_b, v_attention, mask, parent_ids, children):
    ch = children.astype(jnp.int32)
    idx_a = jnp.transpose(ch[:_PA]).reshape(-1)
    idx_b = jnp.transpose(ch[_PA:]).reshape(-1)
    b2 = att_b.reshape(1, ATT)
    v2 = v_attention.reshape(ATT, 1)
    eye_blk = jnp.repeat(jnp.eye(C, dtype=jnp.float32), D, axis=1)

    ranges = [(0, 1800), (1800, 1600), (3400, 1600)]
    gaths = []
    for (q0, nq) in ranges:
        idx_q = jnp.transpose(ch[q0:q0 + nq]).reshape(-1)
        gaths.append(_sc_gather(W_poi, idx_q, nq).reshape(C, nq, D))
    out = None
    for (q0, nq), g in zip(ranges, gaths):
        out = _tc_dense(W_poi, g, att_W, b2, v2, mask, eye_blk, q0, nq,
                        carry=out)
    return _tc_copy_into(W_poi, out)
